# baseline (speedup 1.0000x reference)
import functools

import jax
import jax.numpy as jnp
from jax.experimental import pallas as pl

N = 100000
E = 1600000
A = 1600000
D = 32
G = 4000
L = 8
import numpy as _np
BL_CENTERS = jnp.asarray(_np.arange(0.0, 2.0, 0.1), dtype=jnp.float32)
BA_CENTERS = jnp.asarray(_np.arange(0.0, _np.pi, 0.1), dtype=jnp.float32)


def _dense_body(a_ref, x_ref, s_ref, w1_ref, b1_ref, w2_ref, b2_ref, g_ref,
                b_ref, o_ref, *, last_act):
    a = a_ref[...]
    h = jnp.dot(a, w1_ref[...], preferred_element_type=jnp.float32) + b1_ref[...]
    h = jnp.maximum(h, 0.0)
    h = jnp.dot(h, w2_ref[...], preferred_element_type=jnp.float32) + b2_ref[...]
    mu = jnp.mean(h, axis=-1, keepdims=True)
    var = jnp.mean((h - mu) ** 2, axis=-1, keepdims=True)
    h = (h - mu) * jax.lax.rsqrt(var + 1e-5) * g_ref[...] + b_ref[...]
    h = h * s_ref[...]
    if last_act:
        h = jnp.maximum(h, 0.0)
    o_ref[...] = h + x_ref[...]


@functools.partial(jax.jit, static_argnames=("last_act", "block_rows"))
def _dense_block(aggr, x, scale, p, last_act, block_rows=1024):
    m = aggr.shape[0]
    grid = (pl.cdiv(m, block_rows),)
    row_spec = pl.BlockSpec((block_rows, D), lambda i: (i, 0))
    return pl.pallas_call(
        functools.partial(_dense_body, last_act=last_act),
        grid=grid,
        in_specs=[
            row_spec,
            row_spec,
            pl.BlockSpec((block_rows, 1), lambda i: (i, 0)),
            pl.BlockSpec((D, 2 * D), lambda i: (0, 0)),
            pl.BlockSpec((1, 2 * D), lambda i: (0, 0)),
            pl.BlockSpec((2 * D, D), lambda i: (0, 0)),
            pl.BlockSpec((1, D), lambda i: (0, 0)),
            pl.BlockSpec((1, D), lambda i: (0, 0)),
            pl.BlockSpec((1, D), lambda i: (0, 0)),
        ],
        out_specs=row_spec,
        out_shape=jax.ShapeDtypeStruct((m, D), jnp.float32),
    )(aggr, x, scale, p['W1'], p['b1'].reshape(1, -1), p['W2'],
      p['b2'].reshape(1, -1), p['g'].reshape(1, -1), p['b'].reshape(1, -1))


def _rbf(x, centers, gamma=10.0):
    return jnp.exp(-gamma * (x[:, None] - centers[None, :]) ** 2)


def kernel(params, node_feats, edge_feats, bond_length, bond_angle,
           edge_index, angle_edge_index, batch, edge_batch):
    node_h = 0.0
    for i in range(7):
        node_h = node_h + params['atom_tables'][i][node_feats[:, i]]
    edge_h = 0.0
    for i in range(3):
        edge_h = edge_h + params['bond_tables'][i][edge_feats[:, i]]
    edge_h = edge_h + _rbf(bond_length, BL_CENTERS) @ params['bl_W'] + params['bl_b']
    angle_h = _rbf(bond_angle, BA_CENTERS) @ params['ba_W'] + params['ba_b']

    deg_n = jnp.clip(jnp.bincount(batch, length=G).astype(jnp.float32), 1.0)
    scale_n = jax.lax.rsqrt(deg_n)[batch][:, None]
    deg_e = jnp.clip(jnp.bincount(edge_batch, length=G).astype(jnp.float32), 1.0)
    scale_e = jax.lax.rsqrt(deg_e)[edge_batch][:, None]

    for i in range(L):
        last_act = i < L - 1
        msg_n = node_h[edge_index[0]] + edge_h
        aggr_n = jnp.zeros((N, D), jnp.float32).at[edge_index[1]].add(msg_n)
        msg_e = edge_h[angle_edge_index[0]] + angle_h
        aggr_e = jnp.zeros((E, D), jnp.float32).at[angle_edge_index[1]].add(msg_e)
        node_h = _dense_block(aggr_n, node_h, scale_n, params['atom_blocks'][i], last_act)
        edge_h = _dense_block(aggr_e, edge_h, scale_e, params['bond_blocks'][i], last_act)

    cnt = jnp.clip(jnp.bincount(batch, length=G).astype(node_h.dtype), 1.0)
    graph_repr = jnp.zeros((G, D), node_h.dtype).at[batch].add(node_h) / cnt[:, None]
    return graph_repr


# speedup vs baseline: 1.3727x; 1.3727x over previous
import functools

import jax
import jax.numpy as jnp
from jax import lax
from jax.experimental import pallas as pl
from jax.experimental.pallas import tpu as pltpu
from jax.experimental.pallas import tpu_sc as plsc

N = 100000
E = 1600000
A = 1600000
D = 32
G = 4000
L = 8
import numpy as _np
BL_CENTERS = _np.arange(0.0, 2.0, 0.1).astype(_np.float32)
BA_CENTERS = _np.arange(0.0, _np.pi, 0.1).astype(_np.float32)


def _dense_body(a_ref, x_ref, s_ref, w1_ref, b1_ref, w2_ref, b2_ref, g_ref,
                b_ref, o_ref, *, last_act):
    a = a_ref[...]
    h = jnp.dot(a, w1_ref[...], preferred_element_type=jnp.float32) + b1_ref[...]
    h = jnp.maximum(h, 0.0)
    h = jnp.dot(h, w2_ref[...], preferred_element_type=jnp.float32) + b2_ref[...]
    mu = jnp.mean(h, axis=-1, keepdims=True)
    var = jnp.mean((h - mu) ** 2, axis=-1, keepdims=True)
    h = (h - mu) * jax.lax.rsqrt(var + 1e-5) * g_ref[...] + b_ref[...]
    h = h * s_ref[...]
    if last_act:
        h = jnp.maximum(h, 0.0)
    o_ref[...] = h + x_ref[...]


@functools.partial(jax.jit, static_argnames=("last_act", "block_rows"))
def _dense_block(aggr, x, scale, p, last_act, block_rows=1024):
    m = aggr.shape[0]
    grid = (pl.cdiv(m, block_rows),)
    row_spec = pl.BlockSpec((block_rows, D), lambda i: (i, 0))
    return pl.pallas_call(
        functools.partial(_dense_body, last_act=last_act),
        grid=grid,
        in_specs=[
            row_spec,
            row_spec,
            pl.BlockSpec((block_rows, 1), lambda i: (i, 0)),
            pl.BlockSpec((D, 2 * D), lambda i: (0, 0)),
            pl.BlockSpec((1, 2 * D), lambda i: (0, 0)),
            pl.BlockSpec((2 * D, D), lambda i: (0, 0)),
            pl.BlockSpec((1, D), lambda i: (0, 0)),
            pl.BlockSpec((1, D), lambda i: (0, 0)),
            pl.BlockSpec((1, D), lambda i: (0, 0)),
        ],
        out_specs=row_spec,
        out_shape=jax.ShapeDtypeStruct((m, D), jnp.float32),
    )(aggr, x, scale, p['W1'], p['b1'].reshape(1, -1), p['W2'],
      p['b2'].reshape(1, -1), p['g'].reshape(1, -1), p['b'].reshape(1, -1))


_NH = N // 2
_PAD = 112
_NHP = _NH + _PAD
_SUB = 80
_NSUB = 2
_W = _SUB * _NSUB
_ET = E // 16
_NWIN = _ET // _W
_NGRP = _NWIN // 2

_sc_mesh = plsc.VectorSubcoreMesh(core_axis_name="c", subcore_axis_name="s")


def _node_agg_kernel(nh, eh, srcr, dlf, zr, out, *refs):
    idxs = [[refs[b * _NSUB + g] for g in range(_NSUB)] for b in range(2)]
    o = 2 * _NSUB
    dl = [[refs[o + b * _NSUB + g] for g in range(_NSUB)] for b in range(2)]
    o = 4 * _NSUB
    rows = [[refs[o + b * _NSUB + g] for g in range(_NSUB)] for b in range(2)]
    o = 6 * _NSUB
    erows = [[refs[o + b * _NSUB + g] for g in range(_NSUB)] for b in range(2)]
    spbuf, isem, gsem, ssem = refs[8 * _NSUB:]

    c = lax.axis_index("c")
    s = lax.axis_index("s")

    @pl.when(s == 0)
    def _():
        pltpu.sync_copy(zr, spbuf)
    plsc.subcore_barrier()

    def scat_descs(b):
        ds = []
        for g in range(_NSUB):
            ds.append(pltpu.make_async_copy(rows[b][g], spbuf.at[dl[b][g]],
                                            ssem.at[b]))
            ds.append(pltpu.make_async_copy(erows[b][g], spbuf.at[dl[b][g]],
                                            ssem.at[b]))
        return ds

    def do_window(w, b):
        eoff = s * _ET + w * _W
        ids = []
        for g2 in range(_NSUB):
            ids.append(pltpu.async_copy(
                srcr.at[pl.ds(eoff + g2 * _SUB, _SUB)], idxs[b][g2], isem))
            ids.append(pltpu.async_copy(
                dlf.at[pl.ds(c * E + eoff + g2 * _SUB, _SUB)], dl[b][g2],
                isem))
        for d in ids:
            d.wait()
        gds = []
        for g2 in range(_NSUB):
            gds.append(pltpu.async_copy(nh.at[idxs[b][g2]], rows[b][g2], gsem))
            gds.append(pltpu.async_copy(
                eh.at[pl.ds(eoff + g2 * _SUB, _SUB), :], erows[b][g2], gsem))
        for d in gds:
            d.wait()
        for g2 in range(_NSUB):
            pltpu.async_copy(rows[b][g2], spbuf.at[dl[b][g2]], ssem.at[b],
                             add=True)
            pltpu.async_copy(erows[b][g2], spbuf.at[dl[b][g2]], ssem.at[b],
                             add=True)

    def body(g, carry):
        for b in range(2):
            @pl.when(g > 0)
            def _():
                for d in scat_descs(b):
                    d.wait()

            do_window(g * 2 + b, b)
        return carry

    lax.fori_loop(0, _NGRP, body, 0)
    for d in scat_descs(0):
        d.wait()
    do_window(_NWIN - 1, 0)
    for b in range(2):
        for d in scat_descs(b):
            d.wait()
    plsc.subcore_barrier()

    @pl.when(s == 0)
    def _():
        pltpu.sync_copy(spbuf.at[pl.ds(0, _NH), :],
                        out.at[pl.ds(c * _NH, _NH), :])


@jax.jit
def _node_agg(node_h, edge_h, src, dlflat):
    zeros = jnp.zeros((_NHP, D), jnp.float32)
    scratch = (
        [pltpu.VMEM((_SUB,), jnp.int32) for _ in range(2 * _NSUB)]
        + [pltpu.VMEM((_SUB,), jnp.int32) for _ in range(2 * _NSUB)]
        + [pltpu.VMEM((_SUB, D), jnp.float32) for _ in range(2 * _NSUB)]
        + [pltpu.VMEM((_SUB, D), jnp.float32) for _ in range(2 * _NSUB)]
        + [pltpu.VMEM_SHARED((_NHP, D), jnp.float32),
           pltpu.SemaphoreType.DMA,
           pltpu.SemaphoreType.DMA,
           pltpu.SemaphoreType.DMA((2,))]
    )
    f = pl.kernel(
        _node_agg_kernel,
        out_type=jax.ShapeDtypeStruct((N, D), jnp.float32),
        mesh=_sc_mesh,
        scratch_types=scratch,
        compiler_params=pltpu.CompilerParams(use_tc_tiling_on_sc=False),
    )
    return f(node_h, edge_h, src, dlflat, zeros)


def _make_node_dstloc(dst):
    ar = jnp.arange(E, dtype=jnp.int32) % _PAD
    dl0 = jnp.where(dst < _NH, dst, _NH + ar)
    dl1 = jnp.where(dst >= _NH, dst - _NH, _NH + ar)
    return jnp.concatenate([dl0, dl1]).astype(jnp.int32)


def _rbf(x, centers, gamma=10.0):
    return jnp.exp(-gamma * (x[:, None] - centers[None, :]) ** 2)


def kernel(params, node_feats, edge_feats, bond_length, bond_angle,
           edge_index, angle_edge_index, batch, edge_batch):
    node_h = 0.0
    for i in range(7):
        node_h = node_h + params['atom_tables'][i][node_feats[:, i]]
    edge_h = 0.0
    for i in range(3):
        edge_h = edge_h + params['bond_tables'][i][edge_feats[:, i]]
    edge_h = edge_h + _rbf(bond_length, BL_CENTERS) @ params['bl_W'] + params['bl_b']
    angle_h = _rbf(bond_angle, BA_CENTERS) @ params['ba_W'] + params['ba_b']

    deg_n = jnp.clip(jnp.bincount(batch, length=G).astype(jnp.float32), 1.0)
    scale_n = jax.lax.rsqrt(deg_n)[batch][:, None]
    deg_e = jnp.clip(jnp.bincount(edge_batch, length=G).astype(jnp.float32), 1.0)
    scale_e = jax.lax.rsqrt(deg_e)[edge_batch][:, None]

    src_n = edge_index[0].astype(jnp.int32)
    dlflat_n = _make_node_dstloc(edge_index[1].astype(jnp.int32))

    for i in range(L):
        last_act = i < L - 1
        aggr_n = _node_agg(node_h, edge_h, src_n, dlflat_n)
        msg_e = edge_h[angle_edge_index[0]] + angle_h
        aggr_e = jnp.zeros((E, D), jnp.float32).at[angle_edge_index[1]].add(msg_e)
        node_h = _dense_block(aggr_n, node_h, scale_n, params['atom_blocks'][i], last_act)
        edge_h = _dense_block(aggr_e, edge_h, scale_e, params['bond_blocks'][i], last_act)

    cnt = jnp.clip(jnp.bincount(batch, length=G).astype(node_h.dtype), 1.0)
    graph_repr = jnp.zeros((G, D), node_h.dtype).at[batch].add(node_h) / cnt[:, None]
    return graph_repr


# speedup vs baseline: 2.0142x; 1.4674x over previous
import functools

import jax
import jax.numpy as jnp
from jax import lax
from jax.experimental import pallas as pl
from jax.experimental.pallas import tpu as pltpu
from jax.experimental.pallas import tpu_sc as plsc

N = 100000
E = 1600000
A = 1600000
D = 32
G = 4000
L = 8
import numpy as _np
BL_CENTERS = _np.arange(0.0, 2.0, 0.1).astype(_np.float32)
BA_CENTERS = _np.arange(0.0, _np.pi, 0.1).astype(_np.float32)


def _dense_body(a_ref, x_ref, s_ref, w1_ref, b1_ref, w2_ref, b2_ref, g_ref,
                b_ref, o_ref, *, last_act):
    a = a_ref[...]
    h = jnp.dot(a, w1_ref[...], preferred_element_type=jnp.float32) + b1_ref[...]
    h = jnp.maximum(h, 0.0)
    h = jnp.dot(h, w2_ref[...], preferred_element_type=jnp.float32) + b2_ref[...]
    mu = jnp.mean(h, axis=-1, keepdims=True)
    var = jnp.mean((h - mu) ** 2, axis=-1, keepdims=True)
    h = (h - mu) * jax.lax.rsqrt(var + 1e-5) * g_ref[...] + b_ref[...]
    h = h * s_ref[...]
    if last_act:
        h = jnp.maximum(h, 0.0)
    o_ref[...] = h + x_ref[...]


@functools.partial(jax.jit, static_argnames=("last_act", "block_rows"))
def _dense_block(aggr, x, scale, p, last_act, block_rows=1024):
    m = aggr.shape[0]
    grid = (pl.cdiv(m, block_rows),)
    row_spec = pl.BlockSpec((block_rows, D), lambda i: (i, 0))
    return pl.pallas_call(
        functools.partial(_dense_body, last_act=last_act),
        grid=grid,
        in_specs=[
            row_spec,
            row_spec,
            pl.BlockSpec((block_rows, 1), lambda i: (i, 0)),
            pl.BlockSpec((D, 2 * D), lambda i: (0, 0)),
            pl.BlockSpec((1, 2 * D), lambda i: (0, 0)),
            pl.BlockSpec((2 * D, D), lambda i: (0, 0)),
            pl.BlockSpec((1, D), lambda i: (0, 0)),
            pl.BlockSpec((1, D), lambda i: (0, 0)),
            pl.BlockSpec((1, D), lambda i: (0, 0)),
        ],
        out_specs=row_spec,
        out_shape=jax.ShapeDtypeStruct((m, D), jnp.float32),
    )(aggr, x, scale, p['W1'], p['b1'].reshape(1, -1), p['W2'],
      p['b2'].reshape(1, -1), p['g'].reshape(1, -1), p['b'].reshape(1, -1))


_NH = N // 2
_PAD = 112
_NHP = _NH + _PAD
_SUB = 80
_NSUB = 2
_W = _SUB * _NSUB
_ET = E // 16
_NWIN = _ET // _W
_NGRP = _NWIN // 2

_sc_mesh = plsc.VectorSubcoreMesh(core_axis_name="c", subcore_axis_name="s")


def _node_agg_kernel(nh, eh, srcr, dlf, zr, out, *refs):
    idxs = [[refs[b * _NSUB + g] for g in range(_NSUB)] for b in range(2)]
    o = 2 * _NSUB
    dl = [[refs[o + b * _NSUB + g] for g in range(_NSUB)] for b in range(2)]
    o = 4 * _NSUB
    rows = [[refs[o + b * _NSUB + g] for g in range(_NSUB)] for b in range(2)]
    o = 6 * _NSUB
    erows = [[refs[o + b * _NSUB + g] for g in range(_NSUB)] for b in range(2)]
    spbuf, isem, gsem, ssem = refs[8 * _NSUB:]

    c = lax.axis_index("c")
    s = lax.axis_index("s")

    @pl.when(s == 0)
    def _():
        pltpu.sync_copy(zr, spbuf)
    plsc.subcore_barrier()

    def scat_descs(b):
        ds = []
        for g in range(_NSUB):
            ds.append(pltpu.make_async_copy(rows[b][g], spbuf.at[dl[b][g]],
                                            ssem.at[b]))
            ds.append(pltpu.make_async_copy(erows[b][g], spbuf.at[dl[b][g]],
                                            ssem.at[b]))
        return ds

    def do_window(w, b):
        eoff = s * _ET + w * _W
        ids = []
        for g2 in range(_NSUB):
            ids.append(pltpu.async_copy(
                srcr.at[pl.ds(eoff + g2 * _SUB, _SUB)], idxs[b][g2], isem))
            ids.append(pltpu.async_copy(
                dlf.at[pl.ds(c * E + eoff + g2 * _SUB, _SUB)], dl[b][g2],
                isem))
        for d in ids:
            d.wait()
        gds = []
        for g2 in range(_NSUB):
            gds.append(pltpu.async_copy(nh.at[idxs[b][g2]], rows[b][g2], gsem))
            gds.append(pltpu.async_copy(
                eh.at[pl.ds(eoff + g2 * _SUB, _SUB), :], erows[b][g2], gsem))
        for d in gds:
            d.wait()
        for g2 in range(_NSUB):
            pltpu.async_copy(rows[b][g2], spbuf.at[dl[b][g2]], ssem.at[b],
                             add=True)
            pltpu.async_copy(erows[b][g2], spbuf.at[dl[b][g2]], ssem.at[b],
                             add=True)

    def body(g, carry):
        for b in range(2):
            @pl.when(g > 0)
            def _():
                for d in scat_descs(b):
                    d.wait()

            do_window(g * 2 + b, b)
        return carry

    lax.fori_loop(0, _NGRP, body, 0)
    for d in scat_descs(0):
        d.wait()
    do_window(_NWIN - 1, 0)
    for b in range(2):
        for d in scat_descs(b):
            d.wait()
    plsc.subcore_barrier()

    @pl.when(s == 0)
    def _():
        pltpu.sync_copy(spbuf.at[pl.ds(0, _NH), :],
                        out.at[pl.ds(c * _NH, _NH), :])


@jax.jit
def _node_agg(node_h, edge_h, src, dlflat):
    zeros = jnp.zeros((_NHP, D), jnp.float32)
    scratch = (
        [pltpu.VMEM((_SUB,), jnp.int32) for _ in range(2 * _NSUB)]
        + [pltpu.VMEM((_SUB,), jnp.int32) for _ in range(2 * _NSUB)]
        + [pltpu.VMEM((_SUB, D), jnp.float32) for _ in range(2 * _NSUB)]
        + [pltpu.VMEM((_SUB, D), jnp.float32) for _ in range(2 * _NSUB)]
        + [pltpu.VMEM_SHARED((_NHP, D), jnp.float32),
           pltpu.SemaphoreType.DMA,
           pltpu.SemaphoreType.DMA,
           pltpu.SemaphoreType.DMA((2,))]
    )
    f = pl.kernel(
        _node_agg_kernel,
        out_type=jax.ShapeDtypeStruct((N, D), jnp.float32),
        mesh=_sc_mesh,
        scratch_types=scratch,
        compiler_params=pltpu.CompilerParams(use_tc_tiling_on_sc=False),
    )
    return f(node_h, edge_h, src, dlflat, zeros)


def _make_node_dstloc(dst):
    ar = jnp.arange(E, dtype=jnp.int32) % _PAD
    dl0 = jnp.where(dst < _NH, dst, _NH + ar)
    dl1 = jnp.where(dst >= _NH, dst - _NH, _NH + ar)
    return jnp.concatenate([dl0, dl1]).astype(jnp.int32)


_RC = 50000
_CH = E // _RC
_RCP = _RC + _PAD


def _edge_agg_kernel(table, idxr, dlr, sth, initarr, out, *refs, zero_init):
    idxs = [[refs[b * _NSUB + g] for g in range(_NSUB)] for b in range(2)]
    o = 2 * _NSUB
    dl = [[refs[o + b * _NSUB + g] for g in range(_NSUB)] for b in range(2)]
    o = 4 * _NSUB
    rows = [[refs[o + b * _NSUB + g] for g in range(_NSUB)] for b in range(2)]
    stv, spbuf, isem, gsem, ssem = refs[6 * _NSUB:]

    c = lax.axis_index("c")
    s = lax.axis_index("s")
    dumv = _RC + lax.rem(s * 16 + lax.iota(jnp.int32, 16), _PAD)
    lane = lax.iota(jnp.int32, 16)

    pltpu.sync_copy(sth, stv)

    def rdstart(i):
        acc = jnp.int32(0)
        for j in range(3):
            v = stv[pl.ds(j * 16, 16)]
            acc = acc + jnp.sum(jnp.where(lane + j * 16 == i, v, 0))
        return acc

    def scat_descs(b):
        return [pltpu.make_async_copy(rows[b][g], spbuf.at[dl[b][g]],
                                      ssem.at[b]) for g in range(_NSUB)]

    for k in range(_CH // 2):
        ch = 2 * k + c
        e0 = rdstart(ch)
        e1 = rdstart(ch + 1)
        e0a = (e0 // 8) * 8
        nwin = (e1 - e0a + (_W - 1)) // _W
        nws = (nwin - s + 15) // 16

        for half in range(2):
            @pl.when(s == half)
            def _():
                if zero_init:
                    src_slice = initarr.at[pl.ds(half * (_RC // 2), _RC // 2), :]
                else:
                    src_slice = initarr.at[
                        pl.ds(ch * _RC + half * (_RC // 2), _RC // 2), :]
                pltpu.sync_copy(src_slice,
                                spbuf.at[pl.ds(half * (_RC // 2), _RC // 2), :])

        @pl.when(s == 2)
        def _():
            pltpu.sync_copy(initarr.at[pl.ds(0, _PAD), :],
                            spbuf.at[pl.ds(_RC, _PAD), :])
        plsc.subcore_barrier()

        def do_window(wi, b):
            eoff = e0a + (s + wi * 16) * _W
            ids = []
            for g2 in range(_NSUB):
                ids.append(pltpu.async_copy(
                    idxr.at[pl.ds(eoff + g2 * _SUB, _SUB)], idxs[b][g2], isem))
                ids.append(pltpu.async_copy(
                    dlr.at[pl.ds(eoff + g2 * _SUB, _SUB)], dl[b][g2], isem))
            for d in ids:
                d.wait()
            for g2 in range(_NSUB):
                for v in range(_SUB // 16):
                    pos = eoff + g2 * _SUB + v * 16 + lane
                    ok = (pos >= e0) & (pos < e1)
                    dlv = dl[b][g2][pl.ds(v * 16, 16)]
                    dl[b][g2][pl.ds(v * 16, 16)] = jnp.where(ok, dlv, dumv)
            gds = [pltpu.async_copy(table.at[idxs[b][g2]], rows[b][g2], gsem)
                   for g2 in range(_NSUB)]
            for d in gds:
                d.wait()
            for g2 in range(_NSUB):
                pltpu.async_copy(rows[b][g2], spbuf.at[dl[b][g2]], ssem.at[b],
                                 add=True)

        def body(gi, carry):
            for b in range(2):
                @pl.when(gi > 0)
                def _():
                    for d in scat_descs(b):
                        d.wait()

                do_window(gi * 2 + b, b)
            return carry

        ngrp = nws // 2
        lax.fori_loop(0, ngrp, body, 0)

        @pl.when(nws % 2 == 1)
        def _():
            @pl.when(ngrp > 0)
            def _():
                for d in scat_descs(0):
                    d.wait()
            do_window(ngrp * 2, 0)

        @pl.when(nws >= 1)
        def _():
            for d in scat_descs(0):
                d.wait()

        @pl.when(nws >= 2)
        def _():
            for d in scat_descs(1):
                d.wait()
        plsc.subcore_barrier()
        for half in range(2):
            @pl.when(s == half)
            def _():
                pltpu.sync_copy(
                    spbuf.at[pl.ds(half * (_RC // 2), _RC // 2), :],
                    out.at[pl.ds(ch * _RC + half * (_RC // 2), _RC // 2), :])
        plsc.subcore_barrier()


@functools.partial(jax.jit, static_argnames=("zero_init",))
def _edge_agg(table, idxp, dlp, starts_pad, initarr, zero_init=False):
    scratch = (
        [pltpu.VMEM((_SUB,), jnp.int32) for _ in range(2 * _NSUB)]
        + [pltpu.VMEM((_SUB,), jnp.int32) for _ in range(2 * _NSUB)]
        + [pltpu.VMEM((_SUB, D), jnp.float32) for _ in range(2 * _NSUB)]
        + [pltpu.VMEM((48,), jnp.int32),
           pltpu.VMEM_SHARED((_RCP, D), jnp.float32),
           pltpu.SemaphoreType.DMA,
           pltpu.SemaphoreType.DMA,
           pltpu.SemaphoreType.DMA((2,))]
    )
    f = pl.kernel(
        functools.partial(_edge_agg_kernel, zero_init=zero_init),
        out_type=jax.ShapeDtypeStruct((E, D), jnp.float32),
        mesh=_sc_mesh,
        scratch_types=scratch,
        compiler_params=pltpu.CompilerParams(use_tc_tiling_on_sc=False,
                                             needs_layout_passes=False),
    )
    return f(table, idxp, dlp, starts_pad, initarr)


def _sort_edges(src_a, dst_a):
    iota = jnp.arange(A, dtype=jnp.int32)
    dst_s, src_s, aid_s = lax.sort((dst_a, src_a, iota), num_keys=1)
    starts = jnp.searchsorted(
        dst_s, jnp.arange(_CH + 1, dtype=jnp.int32) * _RC).astype(jnp.int32)
    starts_pad = jnp.concatenate(
        [starts, jnp.full((48 - (_CH + 1),), A, jnp.int32)])
    dls = dst_s - (dst_s // _RC) * _RC
    zpad = jnp.zeros((2 * _W,), jnp.int32)
    return (jnp.concatenate([src_s, zpad]), jnp.concatenate([aid_s, zpad]),
            jnp.concatenate([dls, zpad]), starts_pad)


def _rbf(x, centers, gamma=10.0):
    return jnp.exp(-gamma * (x[:, None] - centers[None, :]) ** 2)


def kernel(params, node_feats, edge_feats, bond_length, bond_angle,
           edge_index, angle_edge_index, batch, edge_batch):
    node_h = 0.0
    for i in range(7):
        node_h = node_h + params['atom_tables'][i][node_feats[:, i]]
    edge_h = 0.0
    for i in range(3):
        edge_h = edge_h + params['bond_tables'][i][edge_feats[:, i]]
    edge_h = edge_h + _rbf(bond_length, BL_CENTERS) @ params['bl_W'] + params['bl_b']
    angle_h = _rbf(bond_angle, BA_CENTERS) @ params['ba_W'] + params['ba_b']

    deg_n = jnp.clip(jnp.bincount(batch, length=G).astype(jnp.float32), 1.0)
    scale_n = jax.lax.rsqrt(deg_n)[batch][:, None]
    deg_e = jnp.clip(jnp.bincount(edge_batch, length=G).astype(jnp.float32), 1.0)
    scale_e = jax.lax.rsqrt(deg_e)[edge_batch][:, None]

    src_n = edge_index[0].astype(jnp.int32)
    dlflat_n = _make_node_dstloc(edge_index[1].astype(jnp.int32))
    src_sp, aid_sp, dls_sp, starts_pad = _sort_edges(
        angle_edge_index[0].astype(jnp.int32),
        angle_edge_index[1].astype(jnp.int32))
    angsum = _edge_agg(angle_h, aid_sp, dls_sp, starts_pad,
                       jnp.zeros((_RC, D), jnp.float32), zero_init=True)

    for i in range(L):
        last_act = i < L - 1
        aggr_n = _node_agg(node_h, edge_h, src_n, dlflat_n)
        aggr_e = _edge_agg(edge_h, src_sp, dls_sp, starts_pad, angsum)
        node_h = _dense_block(aggr_n, node_h, scale_n, params['atom_blocks'][i], last_act)
        edge_h = _dense_block(aggr_e, edge_h, scale_e, params['bond_blocks'][i], last_act)

    cnt = jnp.clip(jnp.bincount(batch, length=G).astype(node_h.dtype), 1.0)
    graph_repr = jnp.zeros((G, D), node_h.dtype).at[batch].add(node_h) / cnt[:, None]
    return graph_repr


# speedup vs baseline: 2.6520x; 1.3166x over previous
import functools

import jax
import jax.numpy as jnp
from jax import lax
from jax.experimental import pallas as pl
from jax.experimental.pallas import tpu as pltpu
from jax.experimental.pallas import tpu_sc as plsc

N = 100000
E = 1600000
A = 1600000
D = 32
G = 4000
L = 8
import numpy as _np
BL_CENTERS = _np.arange(0.0, 2.0, 0.1).astype(_np.float32)
BA_CENTERS = _np.arange(0.0, _np.pi, 0.1).astype(_np.float32)


_SEG = _np.kron(_np.eye(4, dtype=_np.float32), _np.ones((1, D), _np.float32))


def _dense_body(a_ref, x_ref, s_ref, w1_ref, b1_ref, w2_ref, b2_ref, g_ref,
                b_ref, seg_ref, segt_ref, o_ref, *, last_act):
    a = a_ref[...]
    h = jnp.dot(a, w1_ref[...], preferred_element_type=jnp.float32) + b1_ref[...]
    h = jnp.maximum(h, 0.0)
    h = jnp.dot(h, w2_ref[...], preferred_element_type=jnp.float32) + b2_ref[...]
    seg = seg_ref[...]
    segt = segt_ref[...]
    mu = jnp.dot(jnp.dot(h, segt, preferred_element_type=jnp.float32) * (1.0 / D),
                 seg, preferred_element_type=jnp.float32)
    msq = jnp.dot(jnp.dot(h * h, segt, preferred_element_type=jnp.float32) * (1.0 / D),
                  seg, preferred_element_type=jnp.float32)
    var = msq - mu * mu
    h = (h - mu) * jax.lax.rsqrt(var + 1e-5) * g_ref[...] + b_ref[...]
    h = h * jnp.dot(s_ref[...], seg, preferred_element_type=jnp.float32)
    if last_act:
        h = jnp.maximum(h, 0.0)
    o_ref[...] = h + x_ref[...]


@functools.partial(jax.jit, static_argnames=("last_act", "block_rows"))
def _dense_block(aggr, x, scale4, p, last_act, block_rows=512):
    m = aggr.shape[0]
    m4 = m // 4
    a4 = aggr.reshape(m4, 4 * D)
    x4 = x.reshape(m4, 4 * D)
    eye4 = jnp.eye(4, dtype=jnp.float32)
    w1b = jnp.kron(eye4, p['W1'])
    w2b = jnp.kron(eye4, p['W2'])
    b1b = jnp.tile(p['b1'], 4).reshape(1, -1)
    b2b = jnp.tile(p['b2'], 4).reshape(1, -1)
    gb = jnp.tile(p['g'], 4).reshape(1, -1)
    bb = jnp.tile(p['b'], 4).reshape(1, -1)
    seg = jnp.asarray(_SEG)
    segt = seg.T
    grid = (pl.cdiv(m4, block_rows),)
    row_spec = pl.BlockSpec((block_rows, 4 * D), lambda i: (i, 0))
    out = pl.pallas_call(
        functools.partial(_dense_body, last_act=last_act),
        grid=grid,
        in_specs=[
            row_spec,
            row_spec,
            pl.BlockSpec((block_rows, 4), lambda i: (i, 0)),
            pl.BlockSpec((4 * D, 8 * D), lambda i: (0, 0)),
            pl.BlockSpec((1, 8 * D), lambda i: (0, 0)),
            pl.BlockSpec((8 * D, 4 * D), lambda i: (0, 0)),
            pl.BlockSpec((1, 4 * D), lambda i: (0, 0)),
            pl.BlockSpec((1, 4 * D), lambda i: (0, 0)),
            pl.BlockSpec((1, 4 * D), lambda i: (0, 0)),
            pl.BlockSpec((4, 4 * D), lambda i: (0, 0)),
            pl.BlockSpec((4 * D, 4), lambda i: (0, 0)),
        ],
        out_specs=row_spec,
        out_shape=jax.ShapeDtypeStruct((m4, 4 * D), jnp.float32),
    )(a4, x4, scale4, w1b, b1b, w2b, b2b, gb, bb, seg, segt)
    return out.reshape(m, D)


_NH = N // 2
_PAD = 112
_NHP = _NH + _PAD
_SUB = 80
_NSUB = 2
_W = _SUB * _NSUB
_ET = E // 16
_NWIN = _ET // _W
_NGRP = _NWIN // 2

_sc_mesh = plsc.VectorSubcoreMesh(core_axis_name="c", subcore_axis_name="s")


def _node_agg_kernel(nh, eh, srcr, dlf, zr, out, *refs):
    idxs = [[refs[b * _NSUB + g] for g in range(_NSUB)] for b in range(2)]
    o = 2 * _NSUB
    dl = [[refs[o + b * _NSUB + g] for g in range(_NSUB)] for b in range(2)]
    o = 4 * _NSUB
    rows = [[refs[o + b * _NSUB + g] for g in range(_NSUB)] for b in range(2)]
    o = 6 * _NSUB
    erows = [[refs[o + b * _NSUB + g] for g in range(_NSUB)] for b in range(2)]
    spbuf, isem, gsem, ssem = refs[8 * _NSUB:]

    c = lax.axis_index("c")
    s = lax.axis_index("s")

    @pl.when(s == 0)
    def _():
        pltpu.sync_copy(zr, spbuf)
    plsc.subcore_barrier()

    def scat_descs(b):
        ds = []
        for g in range(_NSUB):
            ds.append(pltpu.make_async_copy(rows[b][g], spbuf.at[dl[b][g]],
                                            ssem.at[b]))
            ds.append(pltpu.make_async_copy(erows[b][g], spbuf.at[dl[b][g]],
                                            ssem.at[b]))
        return ds

    def do_window(w, b):
        eoff = s * _ET + w * _W
        ids = []
        for g2 in range(_NSUB):
            ids.append(pltpu.async_copy(
                srcr.at[pl.ds(eoff + g2 * _SUB, _SUB)], idxs[b][g2], isem))
            ids.append(pltpu.async_copy(
                dlf.at[pl.ds(c * E + eoff + g2 * _SUB, _SUB)], dl[b][g2],
                isem))
        for d in ids:
            d.wait()
        gds = []
        for g2 in range(_NSUB):
            gds.append(pltpu.async_copy(nh.at[idxs[b][g2]], rows[b][g2], gsem))
            gds.append(pltpu.async_copy(
                eh.at[pl.ds(eoff + g2 * _SUB, _SUB), :], erows[b][g2], gsem))
        for d in gds:
            d.wait()
        for g2 in range(_NSUB):
            pltpu.async_copy(rows[b][g2], spbuf.at[dl[b][g2]], ssem.at[b],
                             add=True)
            pltpu.async_copy(erows[b][g2], spbuf.at[dl[b][g2]], ssem.at[b],
                             add=True)

    def body(g, carry):
        for b in range(2):
            @pl.when(g > 0)
            def _():
                for d in scat_descs(b):
                    d.wait()

            do_window(g * 2 + b, b)
        return carry

    lax.fori_loop(0, _NGRP, body, 0)
    for d in scat_descs(0):
        d.wait()
    do_window(_NWIN - 1, 0)
    for b in range(2):
        for d in scat_descs(b):
            d.wait()
    plsc.subcore_barrier()

    @pl.when(s == 0)
    def _():
        pltpu.sync_copy(spbuf.at[pl.ds(0, _NH), :],
                        out.at[pl.ds(c * _NH, _NH), :])


@jax.jit
def _node_agg(node_h, edge_h, src, dlflat):
    zeros = jnp.zeros((_NHP, D), jnp.float32)
    scratch = (
        [pltpu.VMEM((_SUB,), jnp.int32) for _ in range(2 * _NSUB)]
        + [pltpu.VMEM((_SUB,), jnp.int32) for _ in range(2 * _NSUB)]
        + [pltpu.VMEM((_SUB, D), jnp.float32) for _ in range(2 * _NSUB)]
        + [pltpu.VMEM((_SUB, D), jnp.float32) for _ in range(2 * _NSUB)]
        + [pltpu.VMEM_SHARED((_NHP, D), jnp.float32),
           pltpu.SemaphoreType.DMA,
           pltpu.SemaphoreType.DMA,
           pltpu.SemaphoreType.DMA((2,))]
    )
    f = pl.kernel(
        _node_agg_kernel,
        out_type=jax.ShapeDtypeStruct((N, D), jnp.float32),
        mesh=_sc_mesh,
        scratch_types=scratch,
        compiler_params=pltpu.CompilerParams(use_tc_tiling_on_sc=False),
    )
    return f(node_h, edge_h, src, dlflat, zeros)


def _make_node_dstloc(dst):
    ar = jnp.arange(E, dtype=jnp.int32) % _PAD
    dl0 = jnp.where(dst < _NH, dst, _NH + ar)
    dl1 = jnp.where(dst >= _NH, dst - _NH, _NH + ar)
    return jnp.concatenate([dl0, dl1]).astype(jnp.int32)


_RC = 50000
_CH = E // _RC
_RCP = _RC + _PAD


def _edge_agg_kernel(table, idxr, dlr, sth, initarr, out, *refs, zero_init):
    idxs = [[refs[b * _NSUB + g] for g in range(_NSUB)] for b in range(2)]
    o = 2 * _NSUB
    dl = [[refs[o + b * _NSUB + g] for g in range(_NSUB)] for b in range(2)]
    o = 4 * _NSUB
    rows = [[refs[o + b * _NSUB + g] for g in range(_NSUB)] for b in range(2)]
    stv, spbuf, isem, gsem, ssem = refs[6 * _NSUB:]

    c = lax.axis_index("c")
    s = lax.axis_index("s")
    dumv = _RC + lax.rem(s * 16 + lax.iota(jnp.int32, 16), _PAD)
    lane = lax.iota(jnp.int32, 16)

    pltpu.sync_copy(sth, stv)

    def rdstart(i):
        acc = jnp.int32(0)
        for j in range(3):
            v = stv[pl.ds(j * 16, 16)]
            acc = acc + jnp.sum(jnp.where(lane + j * 16 == i, v, 0))
        return acc

    def scat_descs(b):
        return [pltpu.make_async_copy(rows[b][g], spbuf.at[dl[b][g]],
                                      ssem.at[b]) for g in range(_NSUB)]

    for k in range(_CH // 2):
        ch = 2 * k + c
        e0 = rdstart(ch)
        e1 = rdstart(ch + 1)
        e0a = (e0 // 8) * 8
        nwin = (e1 - e0a + (_W - 1)) // _W
        nws = (nwin - s + 15) // 16

        for half in range(2):
            @pl.when(s == half)
            def _():
                if zero_init:
                    src_slice = initarr.at[pl.ds(half * (_RC // 2), _RC // 2), :]
                else:
                    src_slice = initarr.at[
                        pl.ds(ch * _RC + half * (_RC // 2), _RC // 2), :]
                pltpu.sync_copy(src_slice,
                                spbuf.at[pl.ds(half * (_RC // 2), _RC // 2), :])

        @pl.when(s == 2)
        def _():
            pltpu.sync_copy(initarr.at[pl.ds(0, _PAD), :],
                            spbuf.at[pl.ds(_RC, _PAD), :])
        plsc.subcore_barrier()

        def do_window(wi, b):
            eoff = e0a + (s + wi * 16) * _W
            ids = []
            for g2 in range(_NSUB):
                ids.append(pltpu.async_copy(
                    idxr.at[pl.ds(eoff + g2 * _SUB, _SUB)], idxs[b][g2], isem))
                ids.append(pltpu.async_copy(
                    dlr.at[pl.ds(eoff + g2 * _SUB, _SUB)], dl[b][g2], isem))
            for d in ids:
                d.wait()
            for g2 in range(_NSUB):
                for v in range(_SUB // 16):
                    pos = eoff + g2 * _SUB + v * 16 + lane
                    ok = (pos >= e0) & (pos < e1)
                    dlv = dl[b][g2][pl.ds(v * 16, 16)]
                    dl[b][g2][pl.ds(v * 16, 16)] = jnp.where(ok, dlv, dumv)
            gds = [pltpu.async_copy(table.at[idxs[b][g2]], rows[b][g2], gsem)
                   for g2 in range(_NSUB)]
            for d in gds:
                d.wait()
            for g2 in range(_NSUB):
                pltpu.async_copy(rows[b][g2], spbuf.at[dl[b][g2]], ssem.at[b],
                                 add=True)

        def body(gi, carry):
            for b in range(2):
                @pl.when(gi > 0)
                def _():
                    for d in scat_descs(b):
                        d.wait()

                do_window(gi * 2 + b, b)
            return carry

        ngrp = nws // 2
        lax.fori_loop(0, ngrp, body, 0)

        @pl.when(nws % 2 == 1)
        def _():
            @pl.when(ngrp > 0)
            def _():
                for d in scat_descs(0):
                    d.wait()
            do_window(ngrp * 2, 0)

        @pl.when(nws >= 1)
        def _():
            for d in scat_descs(0):
                d.wait()

        @pl.when(nws >= 2)
        def _():
            for d in scat_descs(1):
                d.wait()
        plsc.subcore_barrier()
        for half in range(2):
            @pl.when(s == half)
            def _():
                pltpu.sync_copy(
                    spbuf.at[pl.ds(half * (_RC // 2), _RC // 2), :],
                    out.at[pl.ds(ch * _RC + half * (_RC // 2), _RC // 2), :])
        plsc.subcore_barrier()


@functools.partial(jax.jit, static_argnames=("zero_init",))
def _edge_agg(table, idxp, dlp, starts_pad, initarr, zero_init=False):
    scratch = (
        [pltpu.VMEM((_SUB,), jnp.int32) for _ in range(2 * _NSUB)]
        + [pltpu.VMEM((_SUB,), jnp.int32) for _ in range(2 * _NSUB)]
        + [pltpu.VMEM((_SUB, D), jnp.float32) for _ in range(2 * _NSUB)]
        + [pltpu.VMEM((48,), jnp.int32),
           pltpu.VMEM_SHARED((_RCP, D), jnp.float32),
           pltpu.SemaphoreType.DMA,
           pltpu.SemaphoreType.DMA,
           pltpu.SemaphoreType.DMA((2,))]
    )
    f = pl.kernel(
        functools.partial(_edge_agg_kernel, zero_init=zero_init),
        out_type=jax.ShapeDtypeStruct((E, D), jnp.float32),
        mesh=_sc_mesh,
        scratch_types=scratch,
        compiler_params=pltpu.CompilerParams(use_tc_tiling_on_sc=False,
                                             needs_layout_passes=False),
    )
    return f(table, idxp, dlp, starts_pad, initarr)


def _sort_edges(src_a, dst_a):
    iota = jnp.arange(A, dtype=jnp.int32)
    dst_s, src_s, aid_s = lax.sort((dst_a, src_a, iota), num_keys=1)
    starts = jnp.searchsorted(
        dst_s, jnp.arange(_CH + 1, dtype=jnp.int32) * _RC).astype(jnp.int32)
    starts_pad = jnp.concatenate(
        [starts, jnp.full((48 - (_CH + 1),), A, jnp.int32)])
    dls = dst_s - (dst_s // _RC) * _RC
    zpad = jnp.zeros((2 * _W,), jnp.int32)
    return (jnp.concatenate([src_s, zpad]), jnp.concatenate([aid_s, zpad]),
            jnp.concatenate([dls, zpad]), starts_pad)


def _rbf(x, centers, gamma=10.0):
    return jnp.exp(-gamma * (x[:, None] - centers[None, :]) ** 2)


def kernel(params, node_feats, edge_feats, bond_length, bond_angle,
           edge_index, angle_edge_index, batch, edge_batch):
    node_h = 0.0
    for i in range(7):
        node_h = node_h + params['atom_tables'][i][node_feats[:, i]]
    edge_h = 0.0
    for i in range(3):
        edge_h = edge_h + params['bond_tables'][i][edge_feats[:, i]]
    edge_h = edge_h + _rbf(bond_length, BL_CENTERS) @ params['bl_W'] + params['bl_b']
    angle_h = _rbf(bond_angle, BA_CENTERS) @ params['ba_W'] + params['ba_b']

    deg_n = jnp.clip(jnp.bincount(batch, length=G).astype(jnp.float32), 1.0)
    scale_n = jax.lax.rsqrt(deg_n)[batch].reshape(N // 4, 4)
    deg_e = jnp.clip(jnp.bincount(edge_batch, length=G).astype(jnp.float32), 1.0)
    scale_e = jax.lax.rsqrt(deg_e)[edge_batch].reshape(E // 4, 4)

    src_n = edge_index[0].astype(jnp.int32)
    dlflat_n = _make_node_dstloc(edge_index[1].astype(jnp.int32))
    src_sp, aid_sp, dls_sp, starts_pad = _sort_edges(
        angle_edge_index[0].astype(jnp.int32),
        angle_edge_index[1].astype(jnp.int32))
    angsum = _edge_agg(angle_h, aid_sp, dls_sp, starts_pad,
                       jnp.zeros((_RC, D), jnp.float32), zero_init=True)

    for i in range(L):
        last_act = i < L - 1
        aggr_n = _node_agg(node_h, edge_h, src_n, dlflat_n)
        aggr_e = _edge_agg(edge_h, src_sp, dls_sp, starts_pad, angsum)
        node_h = _dense_block(aggr_n, node_h, scale_n, params['atom_blocks'][i], last_act)
        edge_h = _dense_block(aggr_e, edge_h, scale_e, params['bond_blocks'][i], last_act)

    cnt = jnp.clip(jnp.bincount(batch, length=G).astype(node_h.dtype), 1.0)
    graph_repr = jnp.zeros((G, D), node_h.dtype).at[batch].add(node_h) / cnt[:, None]
    return graph_repr


# speedup vs baseline: 2.7772x; 1.0472x over previous
import functools

import jax
import jax.numpy as jnp
from jax import lax
from jax.experimental import pallas as pl
from jax.experimental.pallas import tpu as pltpu
from jax.experimental.pallas import tpu_sc as plsc

N = 100000
E = 1600000
A = 1600000
D = 32
G = 4000
L = 8
import numpy as _np
BL_CENTERS = _np.arange(0.0, 2.0, 0.1).astype(_np.float32)
BA_CENTERS = _np.arange(0.0, _np.pi, 0.1).astype(_np.float32)


_SEG = _np.kron(_np.eye(4, dtype=_np.float32), _np.ones((1, D), _np.float32))


def _dense_body(a_ref, x_ref, s_ref, w1_ref, b1_ref, w2_ref, b2_ref, g_ref,
                b_ref, seg_ref, segt_ref, o_ref, *, last_act):
    a = a_ref[...]
    h = jnp.dot(a, w1_ref[...], preferred_element_type=jnp.float32) + b1_ref[...]
    h = jnp.maximum(h, 0.0)
    h = jnp.dot(h, w2_ref[...], preferred_element_type=jnp.float32) + b2_ref[...]
    seg = seg_ref[...]
    segt = segt_ref[...]
    mu = jnp.dot(jnp.dot(h, segt, preferred_element_type=jnp.float32) * (1.0 / D),
                 seg, preferred_element_type=jnp.float32)
    msq = jnp.dot(jnp.dot(h * h, segt, preferred_element_type=jnp.float32) * (1.0 / D),
                  seg, preferred_element_type=jnp.float32)
    var = msq - mu * mu
    h = (h - mu) * jax.lax.rsqrt(var + 1e-5) * g_ref[...] + b_ref[...]
    h = h * jnp.dot(s_ref[...], seg, preferred_element_type=jnp.float32)
    if last_act:
        h = jnp.maximum(h, 0.0)
    o_ref[...] = h + x_ref[...]


@functools.partial(jax.jit, static_argnames=("last_act", "block_rows"))
def _dense_block(aggr, x, scale4, p, last_act, block_rows=512):
    m = aggr.shape[0]
    m4 = m // 4
    a4 = aggr.reshape(m4, 4 * D)
    x4 = x.reshape(m4, 4 * D)
    eye4 = jnp.eye(4, dtype=jnp.float32)
    w1b = jnp.kron(eye4, p['W1'])
    w2b = jnp.kron(eye4, p['W2'])
    b1b = jnp.tile(p['b1'], 4).reshape(1, -1)
    b2b = jnp.tile(p['b2'], 4).reshape(1, -1)
    gb = jnp.tile(p['g'], 4).reshape(1, -1)
    bb = jnp.tile(p['b'], 4).reshape(1, -1)
    seg = jnp.asarray(_SEG)
    segt = seg.T
    grid = (pl.cdiv(m4, block_rows),)
    row_spec = pl.BlockSpec((block_rows, 4 * D), lambda i: (i, 0))
    out = pl.pallas_call(
        functools.partial(_dense_body, last_act=last_act),
        grid=grid,
        in_specs=[
            row_spec,
            row_spec,
            pl.BlockSpec((block_rows, 4), lambda i: (i, 0)),
            pl.BlockSpec((4 * D, 8 * D), lambda i: (0, 0)),
            pl.BlockSpec((1, 8 * D), lambda i: (0, 0)),
            pl.BlockSpec((8 * D, 4 * D), lambda i: (0, 0)),
            pl.BlockSpec((1, 4 * D), lambda i: (0, 0)),
            pl.BlockSpec((1, 4 * D), lambda i: (0, 0)),
            pl.BlockSpec((1, 4 * D), lambda i: (0, 0)),
            pl.BlockSpec((4, 4 * D), lambda i: (0, 0)),
            pl.BlockSpec((4 * D, 4), lambda i: (0, 0)),
        ],
        out_specs=row_spec,
        out_shape=jax.ShapeDtypeStruct((m4, 4 * D), jnp.float32),
    )(a4, x4, scale4, w1b, b1b, w2b, b2b, gb, bb, seg, segt)
    return out.reshape(m, D)


_NH = N // 2
_PAD = 112
_NHP = _NH + _PAD
_SUB = 80
_NSUB = 2
_W = _SUB * _NSUB
_ET = E // 16
_NWIN = _ET // _W
_NGRP = _NWIN // 2

_sc_mesh = plsc.VectorSubcoreMesh(core_axis_name="c", subcore_axis_name="s")


def _node_agg_kernel(nh, eh, srcr, dlf, zr, out, *refs):
    idxs = [[refs[b * _NSUB + g] for g in range(_NSUB)] for b in range(2)]
    o = 2 * _NSUB
    dl = [[refs[o + b * _NSUB + g] for g in range(_NSUB)] for b in range(2)]
    o = 4 * _NSUB
    rows = [[refs[o + b * _NSUB + g] for g in range(_NSUB)] for b in range(2)]
    o = 6 * _NSUB
    erows = [[refs[o + b * _NSUB + g] for g in range(_NSUB)] for b in range(2)]
    spbuf, isem, gsem, ssem = refs[8 * _NSUB:]

    c = lax.axis_index("c")
    s = lax.axis_index("s")

    @pl.when(s == 0)
    def _():
        pltpu.sync_copy(zr, spbuf)
    plsc.subcore_barrier()

    def scat_descs(b):
        ds = []
        for g in range(_NSUB):
            ds.append(pltpu.make_async_copy(rows[b][g], spbuf.at[dl[b][g]],
                                            ssem.at[b]))
            ds.append(pltpu.make_async_copy(erows[b][g], spbuf.at[dl[b][g]],
                                            ssem.at[b]))
        return ds

    def idx_load(w, b):
        eoff = s * _ET + w * _W
        ids = []
        for g2 in range(_NSUB):
            ids.append(pltpu.async_copy(
                srcr.at[pl.ds(eoff + g2 * _SUB, _SUB)], idxs[b][g2], isem))
            ids.append(pltpu.async_copy(
                dlf.at[pl.ds(c * E + eoff + g2 * _SUB, _SUB)], dl[b][g2],
                isem))
        return ids

    def stage(w, b, first, prefetch):
        eoff = s * _ET + w * _W
        gds = []
        for g2 in range(_NSUB):
            gds.append(pltpu.async_copy(nh.at[idxs[b][g2]], rows[b][g2], gsem))
            gds.append(pltpu.async_copy(
                eh.at[pl.ds(eoff + g2 * _SUB, _SUB), :], erows[b][g2], gsem))
        if first:
            @pl.when(w > 1)
            def _():
                for d in scat_descs(1 - b):
                    d.wait()
        else:
            for d in scat_descs(1 - b):
                d.wait()
        ids = idx_load(w + 1, 1 - b) if prefetch else []
        for d in gds:
            d.wait()
        for g2 in range(_NSUB):
            pltpu.async_copy(rows[b][g2], spbuf.at[dl[b][g2]], ssem.at[b],
                             add=True)
            pltpu.async_copy(erows[b][g2], spbuf.at[dl[b][g2]], ssem.at[b],
                             add=True)
        for d in ids:
            d.wait()

    for d in idx_load(0, 0):
        d.wait()

    def body(g, carry):
        stage(g * 2, 0, True, True)
        stage(g * 2 + 1, 1, False, True)
        return carry

    lax.fori_loop(0, _NGRP, body, 0)
    stage(_NWIN - 1, 0, False, False)
    for d in scat_descs(0):
        d.wait()
    plsc.subcore_barrier()

    @pl.when(s == 0)
    def _():
        pltpu.sync_copy(spbuf.at[pl.ds(0, _NH), :],
                        out.at[pl.ds(c * _NH, _NH), :])


@jax.jit
def _node_agg(node_h, edge_h, src, dlflat):
    zeros = jnp.zeros((_NHP, D), jnp.float32)
    scratch = (
        [pltpu.VMEM((_SUB,), jnp.int32) for _ in range(2 * _NSUB)]
        + [pltpu.VMEM((_SUB,), jnp.int32) for _ in range(2 * _NSUB)]
        + [pltpu.VMEM((_SUB, D), jnp.float32) for _ in range(2 * _NSUB)]
        + [pltpu.VMEM((_SUB, D), jnp.float32) for _ in range(2 * _NSUB)]
        + [pltpu.VMEM_SHARED((_NHP, D), jnp.float32),
           pltpu.SemaphoreType.DMA,
           pltpu.SemaphoreType.DMA,
           pltpu.SemaphoreType.DMA((2,))]
    )
    f = pl.kernel(
        _node_agg_kernel,
        out_type=jax.ShapeDtypeStruct((N, D), jnp.float32),
        mesh=_sc_mesh,
        scratch_types=scratch,
        compiler_params=pltpu.CompilerParams(use_tc_tiling_on_sc=False),
    )
    return f(node_h, edge_h, src, dlflat, zeros)


def _make_node_dstloc(dst):
    ar = jnp.arange(E, dtype=jnp.int32) % _PAD
    dl0 = jnp.where(dst < _NH, dst, _NH + ar)
    dl1 = jnp.where(dst >= _NH, dst - _NH, _NH + ar)
    return jnp.concatenate([dl0, dl1]).astype(jnp.int32)


_RC = 50000
_CH = E // _RC
_RCP = _RC + _PAD


def _edge_agg_kernel(table, idxr, dlr, sth, initarr, out, *refs, zero_init):
    idxs = [[refs[b * _NSUB + g] for g in range(_NSUB)] for b in range(2)]
    o = 2 * _NSUB
    dl = [[refs[o + b * _NSUB + g] for g in range(_NSUB)] for b in range(2)]
    o = 4 * _NSUB
    rows = [[refs[o + b * _NSUB + g] for g in range(_NSUB)] for b in range(2)]
    stv, spbuf, isem, gsem, ssem = refs[6 * _NSUB:]

    c = lax.axis_index("c")
    s = lax.axis_index("s")
    dumv = _RC + lax.rem(s * 16 + lax.iota(jnp.int32, 16), _PAD)
    lane = lax.iota(jnp.int32, 16)

    pltpu.sync_copy(sth, stv)

    def rdstart(i):
        acc = jnp.int32(0)
        for j in range(3):
            v = stv[pl.ds(j * 16, 16)]
            acc = acc + jnp.sum(jnp.where(lane + j * 16 == i, v, 0))
        return acc

    def scat_descs(b):
        return [pltpu.make_async_copy(rows[b][g], spbuf.at[dl[b][g]],
                                      ssem.at[b]) for g in range(_NSUB)]

    for k in range(_CH // 2):
        ch = 2 * k + c
        e0 = rdstart(ch)
        e1 = rdstart(ch + 1)
        e0a = (e0 // 8) * 8
        nwin = (e1 - e0a + (_W - 1)) // _W
        nws = (nwin - s + 15) // 16

        for half in range(2):
            @pl.when(s == half)
            def _():
                if zero_init:
                    src_slice = initarr.at[pl.ds(half * (_RC // 2), _RC // 2), :]
                else:
                    src_slice = initarr.at[
                        pl.ds(ch * _RC + half * (_RC // 2), _RC // 2), :]
                pltpu.sync_copy(src_slice,
                                spbuf.at[pl.ds(half * (_RC // 2), _RC // 2), :])

        @pl.when(s == 2)
        def _():
            pltpu.sync_copy(initarr.at[pl.ds(0, _PAD), :],
                            spbuf.at[pl.ds(_RC, _PAD), :])
        plsc.subcore_barrier()

        def do_window(wi, b):
            eoff = e0a + (s + wi * 16) * _W
            ids = []
            for g2 in range(_NSUB):
                ids.append(pltpu.async_copy(
                    idxr.at[pl.ds(eoff + g2 * _SUB, _SUB)], idxs[b][g2], isem))
                ids.append(pltpu.async_copy(
                    dlr.at[pl.ds(eoff + g2 * _SUB, _SUB)], dl[b][g2], isem))
            for d in ids:
                d.wait()
            for g2 in range(_NSUB):
                for v in range(_SUB // 16):
                    pos = eoff + g2 * _SUB + v * 16 + lane
                    ok = (pos >= e0) & (pos < e1)
                    dlv = dl[b][g2][pl.ds(v * 16, 16)]
                    dl[b][g2][pl.ds(v * 16, 16)] = jnp.where(ok, dlv, dumv)
            gds = [pltpu.async_copy(table.at[idxs[b][g2]], rows[b][g2], gsem)
                   for g2 in range(_NSUB)]
            for d in gds:
                d.wait()
            for g2 in range(_NSUB):
                pltpu.async_copy(rows[b][g2], spbuf.at[dl[b][g2]], ssem.at[b],
                                 add=True)

        def body(gi, carry):
            for b in range(2):
                @pl.when(gi > 0)
                def _():
                    for d in scat_descs(b):
                        d.wait()

                do_window(gi * 2 + b, b)
            return carry

        ngrp = nws // 2
        lax.fori_loop(0, ngrp, body, 0)

        @pl.when(nws % 2 == 1)
        def _():
            @pl.when(ngrp > 0)
            def _():
                for d in scat_descs(0):
                    d.wait()
            do_window(ngrp * 2, 0)

        @pl.when(nws >= 1)
        def _():
            for d in scat_descs(0):
                d.wait()

        @pl.when(nws >= 2)
        def _():
            for d in scat_descs(1):
                d.wait()
        plsc.subcore_barrier()
        for half in range(2):
            @pl.when(s == half)
            def _():
                pltpu.sync_copy(
                    spbuf.at[pl.ds(half * (_RC // 2), _RC // 2), :],
                    out.at[pl.ds(ch * _RC + half * (_RC // 2), _RC // 2), :])
        plsc.subcore_barrier()


@functools.partial(jax.jit, static_argnames=("zero_init",))
def _edge_agg(table, idxp, dlp, starts_pad, initarr, zero_init=False):
    scratch = (
        [pltpu.VMEM((_SUB,), jnp.int32) for _ in range(2 * _NSUB)]
        + [pltpu.VMEM((_SUB,), jnp.int32) for _ in range(2 * _NSUB)]
        + [pltpu.VMEM((_SUB, D), jnp.float32) for _ in range(2 * _NSUB)]
        + [pltpu.VMEM((48,), jnp.int32),
           pltpu.VMEM_SHARED((_RCP, D), jnp.float32),
           pltpu.SemaphoreType.DMA,
           pltpu.SemaphoreType.DMA,
           pltpu.SemaphoreType.DMA((2,))]
    )
    f = pl.kernel(
        functools.partial(_edge_agg_kernel, zero_init=zero_init),
        out_type=jax.ShapeDtypeStruct((E, D), jnp.float32),
        mesh=_sc_mesh,
        scratch_types=scratch,
        compiler_params=pltpu.CompilerParams(use_tc_tiling_on_sc=False,
                                             needs_layout_passes=False),
    )
    return f(table, idxp, dlp, starts_pad, initarr)


def _sort_edges(src_a, dst_a):
    iota = jnp.arange(A, dtype=jnp.int32)
    dst_s, src_s, aid_s = lax.sort((dst_a, src_a, iota), num_keys=1)
    starts = jnp.searchsorted(
        dst_s, jnp.arange(_CH + 1, dtype=jnp.int32) * _RC).astype(jnp.int32)
    starts_pad = jnp.concatenate(
        [starts, jnp.full((48 - (_CH + 1),), A, jnp.int32)])
    dls = dst_s - (dst_s // _RC) * _RC
    zpad = jnp.zeros((2 * _W,), jnp.int32)
    return (jnp.concatenate([src_s, zpad]), jnp.concatenate([aid_s, zpad]),
            jnp.concatenate([dls, zpad]), starts_pad)


def _rbf(x, centers, gamma=10.0):
    return jnp.exp(-gamma * (x[:, None] - centers[None, :]) ** 2)


def kernel(params, node_feats, edge_feats, bond_length, bond_angle,
           edge_index, angle_edge_index, batch, edge_batch):
    node_h = 0.0
    for i in range(7):
        node_h = node_h + params['atom_tables'][i][node_feats[:, i]]
    edge_h = 0.0
    for i in range(3):
        edge_h = edge_h + params['bond_tables'][i][edge_feats[:, i]]
    edge_h = edge_h + _rbf(bond_length, BL_CENTERS) @ params['bl_W'] + params['bl_b']
    angle_h = _rbf(bond_angle, BA_CENTERS) @ params['ba_W'] + params['ba_b']

    deg_n = jnp.clip(jnp.bincount(batch, length=G).astype(jnp.float32), 1.0)
    scale_n = jax.lax.rsqrt(deg_n)[batch].reshape(N // 4, 4)
    deg_e = jnp.clip(jnp.bincount(edge_batch, length=G).astype(jnp.float32), 1.0)
    scale_e = jax.lax.rsqrt(deg_e)[edge_batch].reshape(E // 4, 4)

    src_n = edge_index[0].astype(jnp.int32)
    dlflat_n = _make_node_dstloc(edge_index[1].astype(jnp.int32))
    src_sp, aid_sp, dls_sp, starts_pad = _sort_edges(
        angle_edge_index[0].astype(jnp.int32),
        angle_edge_index[1].astype(jnp.int32))
    angsum = _edge_agg(angle_h, aid_sp, dls_sp, starts_pad,
                       jnp.zeros((_RC, D), jnp.float32), zero_init=True)

    for i in range(L):
        last_act = i < L - 1
        aggr_n = _node_agg(node_h, edge_h, src_n, dlflat_n)
        aggr_e = _edge_agg(edge_h, src_sp, dls_sp, starts_pad, angsum)
        node_h = _dense_block(aggr_n, node_h, scale_n, params['atom_blocks'][i], last_act)
        edge_h = _dense_block(aggr_e, edge_h, scale_e, params['bond_blocks'][i], last_act)

    cnt = jnp.clip(jnp.bincount(batch, length=G).astype(node_h.dtype), 1.0)
    graph_repr = jnp.zeros((G, D), node_h.dtype).at[batch].add(node_h) / cnt[:, None]
    return graph_repr


# speedup vs baseline: 2.8397x; 1.0225x over previous
import functools

import jax
import jax.numpy as jnp
from jax import lax
from jax.experimental import pallas as pl
from jax.experimental.pallas import tpu as pltpu
from jax.experimental.pallas import tpu_sc as plsc

N = 100000
E = 1600000
A = 1600000
D = 32
G = 4000
L = 8
import numpy as _np
BL_CENTERS = _np.arange(0.0, 2.0, 0.1).astype(_np.float32)
BA_CENTERS = _np.arange(0.0, _np.pi, 0.1).astype(_np.float32)


_SEG = _np.kron(_np.eye(4, dtype=_np.float32), _np.ones((1, D), _np.float32))


def _dense_body(a_ref, x_ref, s_ref, w1_ref, b1_ref, w2_ref, b2_ref, g_ref,
                b_ref, seg_ref, segt_ref, o_ref, *, last_act):
    a = a_ref[...]
    h = jnp.dot(a, w1_ref[...], preferred_element_type=jnp.float32) + b1_ref[...]
    h = jnp.maximum(h, 0.0)
    h = jnp.dot(h, w2_ref[...], preferred_element_type=jnp.float32) + b2_ref[...]
    seg = seg_ref[...]
    segt = segt_ref[...]
    mu = jnp.dot(jnp.dot(h, segt, preferred_element_type=jnp.float32) * (1.0 / D),
                 seg, preferred_element_type=jnp.float32)
    msq = jnp.dot(jnp.dot(h * h, segt, preferred_element_type=jnp.float32) * (1.0 / D),
                  seg, preferred_element_type=jnp.float32)
    var = msq - mu * mu
    h = (h - mu) * jax.lax.rsqrt(var + 1e-5) * g_ref[...] + b_ref[...]
    h = h * jnp.dot(s_ref[...], seg, preferred_element_type=jnp.float32)
    if last_act:
        h = jnp.maximum(h, 0.0)
    o_ref[...] = h + x_ref[...]


@functools.partial(jax.jit, static_argnames=("last_act", "block_rows"))
def _dense_block(aggr, x, scale4, p, last_act, block_rows=512):
    m = aggr.shape[0]
    m4 = m // 4
    a4 = aggr.reshape(m4, 4 * D)
    x4 = x.reshape(m4, 4 * D)
    eye4 = jnp.eye(4, dtype=jnp.float32)
    w1b = jnp.kron(eye4, p['W1'])
    w2b = jnp.kron(eye4, p['W2'])
    b1b = jnp.tile(p['b1'], 4).reshape(1, -1)
    b2b = jnp.tile(p['b2'], 4).reshape(1, -1)
    gb = jnp.tile(p['g'], 4).reshape(1, -1)
    bb = jnp.tile(p['b'], 4).reshape(1, -1)
    seg = jnp.asarray(_SEG)
    segt = seg.T
    grid = (pl.cdiv(m4, block_rows),)
    row_spec = pl.BlockSpec((block_rows, 4 * D), lambda i: (i, 0))
    out = pl.pallas_call(
        functools.partial(_dense_body, last_act=last_act),
        grid=grid,
        in_specs=[
            row_spec,
            row_spec,
            pl.BlockSpec((block_rows, 4), lambda i: (i, 0)),
            pl.BlockSpec((4 * D, 8 * D), lambda i: (0, 0)),
            pl.BlockSpec((1, 8 * D), lambda i: (0, 0)),
            pl.BlockSpec((8 * D, 4 * D), lambda i: (0, 0)),
            pl.BlockSpec((1, 4 * D), lambda i: (0, 0)),
            pl.BlockSpec((1, 4 * D), lambda i: (0, 0)),
            pl.BlockSpec((1, 4 * D), lambda i: (0, 0)),
            pl.BlockSpec((4, 4 * D), lambda i: (0, 0)),
            pl.BlockSpec((4 * D, 4), lambda i: (0, 0)),
        ],
        out_specs=row_spec,
        out_shape=jax.ShapeDtypeStruct((m4, 4 * D), jnp.float32),
    )(a4, x4, scale4, w1b, b1b, w2b, b2b, gb, bb, seg, segt)
    return out.reshape(m, D)


_NH = N // 2
_PAD = 112
_NHP = _NH + _PAD
_SUB = 80
_NSUB = 2
_W = _SUB * _NSUB
_ET = E // 16
_NWIN = _ET // _W
_NGRP = _NWIN // 2

_sc_mesh = plsc.VectorSubcoreMesh(core_axis_name="c", subcore_axis_name="s")


def _node_agg_kernel(nh, eh, srcr, dlf, zr, out, *refs):
    idxs = [[refs[b * _NSUB + g] for g in range(_NSUB)] for b in range(2)]
    o = 2 * _NSUB
    dl = [[refs[o + b * _NSUB + g] for g in range(_NSUB)] for b in range(2)]
    o = 4 * _NSUB
    rows = [[refs[o + b * _NSUB + g] for g in range(_NSUB)] for b in range(2)]
    o = 6 * _NSUB
    erows = [[refs[o + b * _NSUB + g] for g in range(_NSUB)] for b in range(2)]
    spbuf, isem, gsem, ssem = refs[8 * _NSUB:]

    c = lax.axis_index("c")
    s = lax.axis_index("s")

    @pl.when(s == 0)
    def _():
        pltpu.sync_copy(zr, spbuf)
    plsc.subcore_barrier()

    def scat_descs(b):
        ds = []
        for g in range(_NSUB):
            ds.append(pltpu.make_async_copy(rows[b][g], spbuf.at[dl[b][g]],
                                            ssem.at[b]))
            ds.append(pltpu.make_async_copy(erows[b][g], spbuf.at[dl[b][g]],
                                            ssem.at[b]))
        return ds

    def idx_load(w, b):
        eoff = s * _ET + w * _W
        ids = []
        for g2 in range(_NSUB):
            ids.append(pltpu.async_copy(
                srcr.at[pl.ds(eoff + g2 * _SUB, _SUB)], idxs[b][g2], isem))
            ids.append(pltpu.async_copy(
                dlf.at[pl.ds(c * E + eoff + g2 * _SUB, _SUB)], dl[b][g2],
                isem))
        return ids

    def stage(w, b, first, prefetch):
        eoff = s * _ET + w * _W
        gds = []
        for g2 in range(_NSUB):
            gds.append(pltpu.async_copy(nh.at[idxs[b][g2]], rows[b][g2], gsem))
            gds.append(pltpu.async_copy(
                eh.at[pl.ds(eoff + g2 * _SUB, _SUB), :], erows[b][g2], gsem))
        if first:
            @pl.when(w > 1)
            def _():
                for d in scat_descs(1 - b):
                    d.wait()
        else:
            for d in scat_descs(1 - b):
                d.wait()
        ids = idx_load(w + 1, 1 - b) if prefetch else []
        for d in gds:
            d.wait()
        for g2 in range(_NSUB):
            pltpu.async_copy(rows[b][g2], spbuf.at[dl[b][g2]], ssem.at[b],
                             add=True)
            pltpu.async_copy(erows[b][g2], spbuf.at[dl[b][g2]], ssem.at[b],
                             add=True)
        for d in ids:
            d.wait()

    for d in idx_load(0, 0):
        d.wait()

    def body(g, carry):
        stage(g * 2, 0, True, True)
        stage(g * 2 + 1, 1, False, True)
        return carry

    lax.fori_loop(0, _NGRP, body, 0)
    stage(_NWIN - 1, 0, False, False)
    for d in scat_descs(0):
        d.wait()
    plsc.subcore_barrier()

    @pl.when(s == 0)
    def _():
        pltpu.sync_copy(spbuf.at[pl.ds(0, _NH), :],
                        out.at[pl.ds(c * _NH, _NH), :])


@jax.jit
def _node_agg(node_h, edge_h, src, dlflat):
    zeros = jnp.zeros((_NHP, D), jnp.float32)
    scratch = (
        [pltpu.VMEM((_SUB,), jnp.int32) for _ in range(2 * _NSUB)]
        + [pltpu.VMEM((_SUB,), jnp.int32) for _ in range(2 * _NSUB)]
        + [pltpu.VMEM((_SUB, D), jnp.float32) for _ in range(2 * _NSUB)]
        + [pltpu.VMEM((_SUB, D), jnp.float32) for _ in range(2 * _NSUB)]
        + [pltpu.VMEM_SHARED((_NHP, D), jnp.float32),
           pltpu.SemaphoreType.DMA,
           pltpu.SemaphoreType.DMA,
           pltpu.SemaphoreType.DMA((2,))]
    )
    f = pl.kernel(
        _node_agg_kernel,
        out_type=jax.ShapeDtypeStruct((N, D), jnp.float32),
        mesh=_sc_mesh,
        scratch_types=scratch,
        compiler_params=pltpu.CompilerParams(use_tc_tiling_on_sc=False),
    )
    return f(node_h, edge_h, src, dlflat, zeros)


def _make_node_dstloc(dst):
    ar = jnp.arange(E, dtype=jnp.int32) % _PAD
    dl0 = jnp.where(dst < _NH, dst, _NH + ar)
    dl1 = jnp.where(dst >= _NH, dst - _NH, _NH + ar)
    return jnp.concatenate([dl0, dl1]).astype(jnp.int32)


_RC = 50000
_CH = E // _RC
_RCP = _RC + _PAD
_NSUBE = 4
_WE = _SUB * _NSUBE


def _edge_agg_kernel(table, idxr, dlr, sth, initarr, out, *refs, zero_init):
    idxs = [[refs[b * _NSUBE + g] for g in range(_NSUBE)] for b in range(2)]
    o = 2 * _NSUBE
    dl = [[refs[o + b * _NSUBE + g] for g in range(_NSUBE)] for b in range(2)]
    o = 4 * _NSUBE
    rows = [[refs[o + b * _NSUBE + g] for g in range(_NSUBE)] for b in range(2)]
    stv, spbuf, isem, gsem, ssem = refs[6 * _NSUBE:]

    c = lax.axis_index("c")
    s = lax.axis_index("s")
    dumv = _RC + lax.rem(s * 16 + lax.iota(jnp.int32, 16), _PAD)
    lane = lax.iota(jnp.int32, 16)

    pltpu.sync_copy(sth, stv)

    def rdstart(i):
        acc = jnp.int32(0)
        for j in range(3):
            v = stv[pl.ds(j * 16, 16)]
            acc = acc + jnp.sum(jnp.where(lane + j * 16 == i, v, 0))
        return acc

    def scat_descs(b):
        return [pltpu.make_async_copy(rows[b][g], spbuf.at[dl[b][g]],
                                      ssem.at[b]) for g in range(_NSUBE)]

    for k in range(_CH // 2):
        ch = 2 * k + c
        e0 = rdstart(ch)
        e1 = rdstart(ch + 1)
        e0a = (e0 // 8) * 8
        nwin = (e1 - e0a + (_WE - 1)) // _WE
        nws = (nwin - s + 15) // 16

        for half in range(2):
            @pl.when(s == half)
            def _():
                if zero_init:
                    src_slice = initarr.at[pl.ds(half * (_RC // 2), _RC // 2), :]
                else:
                    src_slice = initarr.at[
                        pl.ds(ch * _RC + half * (_RC // 2), _RC // 2), :]
                pltpu.sync_copy(src_slice,
                                spbuf.at[pl.ds(half * (_RC // 2), _RC // 2), :])

        @pl.when(s == 2)
        def _():
            pltpu.sync_copy(initarr.at[pl.ds(0, _PAD), :],
                            spbuf.at[pl.ds(_RC, _PAD), :])
        plsc.subcore_barrier()

        def do_window(wi, b):
            eoff = e0a + (s + wi * 16) * _WE
            ids = []
            for g2 in range(_NSUBE):
                ids.append(pltpu.async_copy(
                    idxr.at[pl.ds(eoff + g2 * _SUB, _SUB)], idxs[b][g2], isem))
                ids.append(pltpu.async_copy(
                    dlr.at[pl.ds(eoff + g2 * _SUB, _SUB)], dl[b][g2], isem))
            for d in ids:
                d.wait()
            for g2 in range(_NSUBE):
                for v in range(_SUB // 16):
                    pos = eoff + g2 * _SUB + v * 16 + lane
                    ok = (pos >= e0) & (pos < e1)
                    dlv = dl[b][g2][pl.ds(v * 16, 16)]
                    dl[b][g2][pl.ds(v * 16, 16)] = jnp.where(ok, dlv, dumv)
            gds = [pltpu.async_copy(table.at[idxs[b][g2]], rows[b][g2], gsem)
                   for g2 in range(_NSUBE)]
            for d in gds:
                d.wait()
            for g2 in range(_NSUBE):
                pltpu.async_copy(rows[b][g2], spbuf.at[dl[b][g2]], ssem.at[b],
                                 add=True)

        def body(gi, carry):
            for b in range(2):
                @pl.when(gi > 0)
                def _():
                    for d in scat_descs(b):
                        d.wait()

                do_window(gi * 2 + b, b)
            return carry

        ngrp = nws // 2
        lax.fori_loop(0, ngrp, body, 0)

        @pl.when(nws % 2 == 1)
        def _():
            @pl.when(ngrp > 0)
            def _():
                for d in scat_descs(0):
                    d.wait()
            do_window(ngrp * 2, 0)

        @pl.when(nws >= 1)
        def _():
            for d in scat_descs(0):
                d.wait()

        @pl.when(nws >= 2)
        def _():
            for d in scat_descs(1):
                d.wait()
        plsc.subcore_barrier()
        for half in range(2):
            @pl.when(s == half)
            def _():
                pltpu.sync_copy(
                    spbuf.at[pl.ds(half * (_RC // 2), _RC // 2), :],
                    out.at[pl.ds(ch * _RC + half * (_RC // 2), _RC // 2), :])
        plsc.subcore_barrier()


@functools.partial(jax.jit, static_argnames=("zero_init",))
def _edge_agg(table, idxp, dlp, starts_pad, initarr, zero_init=False):
    scratch = (
        [pltpu.VMEM((_SUB,), jnp.int32) for _ in range(2 * _NSUBE)]
        + [pltpu.VMEM((_SUB,), jnp.int32) for _ in range(2 * _NSUBE)]
        + [pltpu.VMEM((_SUB, D), jnp.float32) for _ in range(2 * _NSUBE)]
        + [pltpu.VMEM((48,), jnp.int32),
           pltpu.VMEM_SHARED((_RCP, D), jnp.float32),
           pltpu.SemaphoreType.DMA,
           pltpu.SemaphoreType.DMA,
           pltpu.SemaphoreType.DMA((2,))]
    )
    f = pl.kernel(
        functools.partial(_edge_agg_kernel, zero_init=zero_init),
        out_type=jax.ShapeDtypeStruct((E, D), jnp.float32),
        mesh=_sc_mesh,
        scratch_types=scratch,
        compiler_params=pltpu.CompilerParams(use_tc_tiling_on_sc=False,
                                             needs_layout_passes=False),
    )
    return f(table, idxp, dlp, starts_pad, initarr)


def _sort_edges(src_a, dst_a):
    iota = jnp.arange(A, dtype=jnp.int32)
    dst_s, src_s, aid_s = lax.sort((dst_a, src_a, iota), num_keys=1)
    starts = jnp.searchsorted(
        dst_s, jnp.arange(_CH + 1, dtype=jnp.int32) * _RC).astype(jnp.int32)
    starts_pad = jnp.concatenate(
        [starts, jnp.full((48 - (_CH + 1),), A, jnp.int32)])
    dls = dst_s - (dst_s // _RC) * _RC
    zpad = jnp.zeros((2 * _W,), jnp.int32)
    return (jnp.concatenate([src_s, zpad]), jnp.concatenate([aid_s, zpad]),
            jnp.concatenate([dls, zpad]), starts_pad)


def _rbf(x, centers, gamma=10.0):
    return jnp.exp(-gamma * (x[:, None] - centers[None, :]) ** 2)


def kernel(params, node_feats, edge_feats, bond_length, bond_angle,
           edge_index, angle_edge_index, batch, edge_batch):
    node_h = 0.0
    for i in range(7):
        node_h = node_h + params['atom_tables'][i][node_feats[:, i]]
    edge_h = 0.0
    for i in range(3):
        edge_h = edge_h + params['bond_tables'][i][edge_feats[:, i]]
    edge_h = edge_h + _rbf(bond_length, BL_CENTERS) @ params['bl_W'] + params['bl_b']
    angle_h = _rbf(bond_angle, BA_CENTERS) @ params['ba_W'] + params['ba_b']

    deg_n = jnp.clip(jnp.bincount(batch, length=G).astype(jnp.float32), 1.0)
    scale_n = jax.lax.rsqrt(deg_n)[batch].reshape(N // 4, 4)
    deg_e = jnp.clip(jnp.bincount(edge_batch, length=G).astype(jnp.float32), 1.0)
    scale_e = jax.lax.rsqrt(deg_e)[edge_batch].reshape(E // 4, 4)

    src_n = edge_index[0].astype(jnp.int32)
    dlflat_n = _make_node_dstloc(edge_index[1].astype(jnp.int32))
    src_sp, aid_sp, dls_sp, starts_pad = _sort_edges(
        angle_edge_index[0].astype(jnp.int32),
        angle_edge_index[1].astype(jnp.int32))
    angsum = _edge_agg(angle_h, aid_sp, dls_sp, starts_pad,
                       jnp.zeros((_RC, D), jnp.float32), zero_init=True)

    for i in range(L):
        last_act = i < L - 1
        aggr_n = _node_agg(node_h, edge_h, src_n, dlflat_n)
        aggr_e = _edge_agg(edge_h, src_sp, dls_sp, starts_pad, angsum)
        node_h = _dense_block(aggr_n, node_h, scale_n, params['atom_blocks'][i], last_act)
        edge_h = _dense_block(aggr_e, edge_h, scale_e, params['bond_blocks'][i], last_act)

    cnt = jnp.clip(jnp.bincount(batch, length=G).astype(node_h.dtype), 1.0)
    graph_repr = jnp.zeros((G, D), node_h.dtype).at[batch].add(node_h) / cnt[:, None]
    return graph_repr
